# baseline (speedup 1.0000x reference)
import functools

import jax
import jax.numpy as jnp
from jax import lax
from jax.experimental import pallas as pl
from jax.experimental.pallas import tpu as pltpu
from jax.experimental.pallas import tpu_sc as plsc

_CHUNK = 64
_NBUF = 2


@functools.cache
def _build(B, C, D_mu, D_al):
    info = plsc.get_sparse_core_info()
    NC, NS = info.num_cores, info.num_subcores
    NW = NC * NS
    b_per_w = B // NW
    assert B % (NW * _CHUNK) == 0
    n_chunks = b_per_w // _CHUNK

    mesh = plsc.VectorSubcoreMesh(core_axis_name="c", subcore_axis_name="s")

    @functools.partial(
        pl.kernel,
        mesh=mesh,
        out_type=[
            jax.ShapeDtypeStruct((B, D_mu), jnp.float32),
            jax.ShapeDtypeStruct((B, D_mu), jnp.float32),
            jax.ShapeDtypeStruct((B, D_al), jnp.float32),
        ],
        scratch_types=[
            pltpu.VMEM((n_chunks, _CHUNK), jnp.int32),
            pltpu.VMEM((_NBUF, _CHUNK, D_mu), jnp.float32),
            pltpu.VMEM((_NBUF, _CHUNK, D_mu), jnp.float32),
            pltpu.VMEM((_NBUF, _CHUNK, D_al), jnp.float32),
            pltpu.SemaphoreType.DMA,
            pltpu.SemaphoreType.DMA,
        ],
    )
    def gather_kernel(ids_hbm, mu_hbm, sg_hbm, al_hbm,
                      mu_out, sg_out, al_out,
                      idx_v, mu_v, sg_v, al_v, sem_g, sem_w):
        wid = lax.axis_index("s") * NC + lax.axis_index("c")
        pltpu.sync_copy(ids_hbm.at[pl.ds(wid * n_chunks, n_chunks)], idx_v)

        def start_gather(j):
            b = j % _NBUF
            row_ids = idx_v.at[j]
            return (pltpu.async_copy(mu_hbm.at[row_ids], mu_v.at[b], sem_g),
                    pltpu.async_copy(sg_hbm.at[row_ids], sg_v.at[b], sem_g),
                    pltpu.async_copy(al_hbm.at[row_ids], al_v.at[b], sem_g))

        gather_cps = [None] * _NBUF
        write_cps = [None] * _NBUF
        gather_cps[0] = start_gather(0)
        for j in range(n_chunks):
            b = j % _NBUF
            for cp in gather_cps[b]:
                cp.wait()
            base = (wid * n_chunks + j) * _CHUNK
            write_cps[b] = (
                pltpu.async_copy(mu_v.at[b], mu_out.at[pl.ds(base, _CHUNK)], sem_w),
                pltpu.async_copy(sg_v.at[b], sg_out.at[pl.ds(base, _CHUNK)], sem_w),
                pltpu.async_copy(al_v.at[b], al_out.at[pl.ds(base, _CHUNK)], sem_w),
            )
            if j + 1 < n_chunks:
                nb = (j + 1) % _NBUF
                if write_cps[nb] is not None:
                    for cp in write_cps[nb]:
                        cp.wait()
                    write_cps[nb] = None
                gather_cps[nb] = start_gather(j + 1)
        for b in range(_NBUF):
            if write_cps[b] is not None:
                for cp in write_cps[b]:
                    cp.wait()

    return gather_kernel


def kernel(category_ids, mu, sigma, alpha):
    B = category_ids.shape[0]
    C, K, _ = mu.shape
    D_mu = K * 3
    D_al = K * alpha.shape[2]
    ids2 = category_ids.astype(jnp.int32).reshape(B // _CHUNK, _CHUNK)
    mu2 = mu.reshape(C, D_mu)
    sg2 = sigma.reshape(C, D_mu)
    al2 = alpha.reshape(C, D_al)
    f = _build(B, C, D_mu, D_al)
    mu_o, sg_o, al_o = f(ids2, mu2, sg2, al2)
    return (mu_o.reshape(B, K, 3), sg_o.reshape(B, K, 3),
            al_o.reshape(B, K, alpha.shape[2]))


# speedup vs baseline: 2.8651x; 2.8651x over previous
import functools

import jax
import jax.numpy as jnp
from jax import lax
from jax.experimental import pallas as pl
from jax.experimental.pallas import tpu as pltpu
from jax.experimental.pallas import tpu_sc as plsc

_CHUNK = 64
_NBUF = 2


@functools.cache
def _build(B, C, K, P_mu, P_al):
    info = plsc.get_sparse_core_info()
    NC, NS = info.num_cores, info.num_subcores
    NW = NC * NS
    b_per_w = B // NW
    assert B % (NW * _CHUNK) == 0
    n_chunks = b_per_w // _CHUNK

    mesh = plsc.VectorSubcoreMesh(core_axis_name="c", subcore_axis_name="s")

    @functools.partial(
        pl.kernel,
        mesh=mesh,
        out_type=[
            jax.ShapeDtypeStruct((P_mu, B, K), jnp.float32),
            jax.ShapeDtypeStruct((P_mu, B, K), jnp.float32),
            jax.ShapeDtypeStruct((P_al, B, K), jnp.float32),
        ],
        scratch_types=[
            pltpu.VMEM((n_chunks, P_mu, _CHUNK), jnp.int32),
            pltpu.VMEM((_NBUF, P_mu, _CHUNK, K), jnp.float32),
            pltpu.VMEM((_NBUF, P_mu, _CHUNK, K), jnp.float32),
            pltpu.VMEM((_NBUF, P_al, _CHUNK, K), jnp.float32),
            pltpu.SemaphoreType.DMA,
            pltpu.SemaphoreType.DMA,
        ],
    )
    def gather_kernel(ids_hbm, mu_hbm, sg_hbm, al_hbm,
                      mu_out, sg_out, al_out,
                      idx_v, mu_v, sg_v, al_v, sem_g, sem_w):
        wid = lax.axis_index("s") * NC + lax.axis_index("c")
        pltpu.sync_copy(ids_hbm.at[pl.ds(wid * n_chunks, n_chunks)], idx_v)

        def start_gather(j):
            b = j % _NBUF
            cps = []
            for p in range(P_mu):
                row_ids = idx_v.at[j, p]
                cps.append(pltpu.async_copy(
                    mu_hbm.at[row_ids], mu_v.at[b, p], sem_g))
                cps.append(pltpu.async_copy(
                    sg_hbm.at[row_ids], sg_v.at[b, p], sem_g))
            for p in range(P_al):
                cps.append(pltpu.async_copy(
                    al_hbm.at[idx_v.at[j, p]], al_v.at[b, p], sem_g))
            return cps

        def start_write(j):
            b = j % _NBUF
            base = (wid * n_chunks + j) * _CHUNK
            cps = []
            for p in range(P_mu):
                cps.append(pltpu.async_copy(
                    mu_v.at[b, p], mu_out.at[p, pl.ds(base, _CHUNK)], sem_w))
                cps.append(pltpu.async_copy(
                    sg_v.at[b, p], sg_out.at[p, pl.ds(base, _CHUNK)], sem_w))
            for p in range(P_al):
                cps.append(pltpu.async_copy(
                    al_v.at[b, p], al_out.at[p, pl.ds(base, _CHUNK)], sem_w))
            return cps

        gather_cps = [None] * _NBUF
        write_cps = [None] * _NBUF
        gather_cps[0] = start_gather(0)
        for j in range(n_chunks):
            b = j % _NBUF
            for cp in gather_cps[b]:
                cp.wait()
            write_cps[b] = start_write(j)
            if j + 1 < n_chunks:
                nb = (j + 1) % _NBUF
                if write_cps[nb] is not None:
                    for cp in write_cps[nb]:
                        cp.wait()
                    write_cps[nb] = None
                gather_cps[nb] = start_gather(j + 1)
        for b in range(_NBUF):
            if write_cps[b] is not None:
                for cp in write_cps[b]:
                    cp.wait()

    return gather_kernel


def kernel(category_ids, mu, sigma, alpha):
    B = category_ids.shape[0]
    C, K, P_mu = mu.shape
    P_al = alpha.shape[2]
    ids = category_ids.astype(jnp.int32)
    ids3 = ids[:, None] + jnp.arange(P_mu, dtype=jnp.int32)[None, :] * C
    ids3 = jnp.transpose(ids3.reshape(B // _CHUNK, _CHUNK, P_mu), (0, 2, 1))
    mu_p = jnp.transpose(mu, (2, 0, 1)).reshape(P_mu * C, K)
    sg_p = jnp.transpose(sigma, (2, 0, 1)).reshape(P_mu * C, K)
    al_p = jnp.transpose(alpha, (2, 0, 1)).reshape(P_al * C, K)
    f = _build(B, C, K, P_mu, P_al)
    mu_o, sg_o, al_o = f(ids3, mu_p, sg_p, al_p)
    return (jnp.transpose(mu_o, (1, 2, 0)),
            jnp.transpose(sg_o, (1, 2, 0)),
            jnp.transpose(al_o, (1, 2, 0)))


# speedup vs baseline: 3.0795x; 1.0748x over previous
import functools

import jax
import jax.numpy as jnp
from jax import lax
from jax.experimental import pallas as pl
from jax.experimental.pallas import tpu as pltpu
from jax.experimental.pallas import tpu_sc as plsc

_CHUNK = 128
_NBUF = 4


@functools.cache
def _build(B, C, K, P_mu, P_al):
    info = plsc.get_sparse_core_info()
    NC, NS = info.num_cores, info.num_subcores
    NW = NC * NS
    b_per_w = B // NW
    assert B % (NW * _CHUNK) == 0
    n_chunks = b_per_w // _CHUNK

    mesh = plsc.VectorSubcoreMesh(core_axis_name="c", subcore_axis_name="s")

    @functools.partial(
        pl.kernel,
        mesh=mesh,
        out_type=[
            jax.ShapeDtypeStruct((P_mu, B, K), jnp.float32),
            jax.ShapeDtypeStruct((P_mu, B, K), jnp.float32),
            jax.ShapeDtypeStruct((P_al, B, K), jnp.float32),
        ],
        scratch_types=[
            pltpu.VMEM((n_chunks, P_mu, _CHUNK), jnp.int32),
            pltpu.VMEM((_NBUF, _CHUNK, K), jnp.float32),
            pltpu.SemaphoreType.DMA((_NBUF,)),
            pltpu.SemaphoreType.DMA((_NBUF,)),
        ],
    )
    def gather_kernel(ids_hbm, mu_hbm, sg_hbm, al_hbm,
                      mu_out, sg_out, al_out,
                      idx_v, buf_v, sem_g, sem_w):
        wid = lax.axis_index("s") * NC + lax.axis_index("c")
        pltpu.sync_copy(ids_hbm.at[pl.ds(wid * n_chunks, n_chunks)], idx_v)

        tasks = []
        for j in range(n_chunks):
            for p in range(P_mu):
                tasks.append((mu_hbm, mu_out, p, j))
                tasks.append((sg_hbm, sg_out, p, j))
            for p in range(P_al):
                tasks.append((al_hbm, al_out, p, j))

        def start_gather(t):
            tbl, _, p, j = tasks[t]
            b = t % _NBUF
            return pltpu.async_copy(
                tbl.at[idx_v.at[j, p]], buf_v.at[b], sem_g.at[b])

        def start_write(t):
            _, out, p, j = tasks[t]
            b = t % _NBUF
            base = (wid * n_chunks + j) * _CHUNK
            return pltpu.async_copy(
                buf_v.at[b], out.at[p, pl.ds(base, _CHUNK)], sem_w.at[b])

        n_tasks = len(tasks)
        gather_cps = [None] * _NBUF
        write_cps = [None] * _NBUF
        for t in range(min(_NBUF - 1, n_tasks)):
            gather_cps[t % _NBUF] = start_gather(t)
        for t in range(n_tasks):
            b = t % _NBUF
            gather_cps[b].wait()
            write_cps[b] = start_write(t)
            nxt = t + _NBUF - 1
            if nxt < n_tasks:
                nb = nxt % _NBUF
                if write_cps[nb] is not None:
                    write_cps[nb].wait()
                    write_cps[nb] = None
                gather_cps[nb] = start_gather(nxt)
        for b in range(_NBUF):
            if write_cps[b] is not None:
                write_cps[b].wait()

    return gather_kernel


def kernel(category_ids, mu, sigma, alpha):
    B = category_ids.shape[0]
    C, K, P_mu = mu.shape
    P_al = alpha.shape[2]
    ids = category_ids.astype(jnp.int32)
    ids3 = ids[:, None] + jnp.arange(P_mu, dtype=jnp.int32)[None, :] * C
    ids3 = jnp.transpose(ids3.reshape(B // _CHUNK, _CHUNK, P_mu), (0, 2, 1))
    mu_p = jnp.transpose(mu, (2, 0, 1)).reshape(P_mu * C, K)
    sg_p = jnp.transpose(sigma, (2, 0, 1)).reshape(P_mu * C, K)
    al_p = jnp.transpose(alpha, (2, 0, 1)).reshape(P_al * C, K)
    f = _build(B, C, K, P_mu, P_al)
    mu_o, sg_o, al_o = f(ids3, mu_p, sg_p, al_p)
    return (jnp.transpose(mu_o, (1, 2, 0)),
            jnp.transpose(sg_o, (1, 2, 0)),
            jnp.transpose(al_o, (1, 2, 0)))


# speedup vs baseline: 4.1463x; 1.3464x over previous
import functools

import jax
import jax.numpy as jnp
from jax import lax
from jax.experimental import pallas as pl
from jax.experimental.pallas import tpu as pltpu
from jax.experimental.pallas import tpu_sc as plsc

_CHUNK = 128
_NBUF = 4


@functools.cache
def _build(B, C, K, P_mu, P_al):
    info = plsc.get_sparse_core_info()
    NC, NS = info.num_cores, info.num_subcores
    NW = NC * NS
    b_per_w = B // NW
    assert B % (NW * _CHUNK) == 0
    n_chunks = b_per_w // _CHUNK
    n_planes = 2 * P_mu + P_al
    R_mu = P_mu * C
    R_al = P_al * C

    mesh = plsc.VectorSubcoreMesh(core_axis_name="c", subcore_axis_name="s")

    @functools.partial(
        pl.kernel,
        mesh=mesh,
        out_type=[
            jax.ShapeDtypeStruct((P_mu, B, K), jnp.float32),
            jax.ShapeDtypeStruct((P_mu, B, K), jnp.float32),
            jax.ShapeDtypeStruct((P_al, B, K), jnp.float32),
        ],
        scratch_types=[
            pltpu.VMEM((n_chunks, n_planes, _CHUNK), jnp.int32),
            pltpu.VMEM((_NBUF, _CHUNK, K), jnp.float32),
            pltpu.VMEM_SHARED((2 * R_mu + R_al, K), jnp.float32),
            pltpu.SemaphoreType.DMA((_NBUF,)),
            pltpu.SemaphoreType.DMA((_NBUF,)),
        ],
    )
    def gather_kernel(ids_hbm, mu_hbm, sg_hbm, al_hbm,
                      mu_out, sg_out, al_out,
                      idx_v, buf_v, spm, sem_g, sem_w):
        cid = lax.axis_index("c")
        sid = lax.axis_index("s")
        wid = sid * NC + cid

        half = NS // 2

        def _stage(src, rows, dst_base, lane):
            per = ((rows + half - 1) // half + 7) // 8 * 8
            rem = rows - per * (half - 1)
            assert rem > 0 and rem % 8 == 0

            @pl.when((lane >= 0) & (lane < half - 1))
            def _bulk():
                pltpu.sync_copy(
                    src.at[pl.ds(lane * per, per)],
                    spm.at[pl.ds(dst_base + lane * per, per)])

            @pl.when(lane == half - 1)
            def _tail():
                off = per * (half - 1)
                pltpu.sync_copy(
                    src.at[pl.ds(off, rem)],
                    spm.at[pl.ds(dst_base + off, rem)])

        _stage(mu_hbm, R_mu, 0, sid)
        _stage(sg_hbm, R_mu, R_mu, sid - half)
        _stage(al_hbm, R_al, 2 * R_mu, sid)

        cp_idx = pltpu.async_copy(
            ids_hbm.at[pl.ds(wid * n_chunks, n_chunks)], idx_v, sem_w.at[0])
        plsc.subcore_barrier()
        cp_idx.wait()

        outs = ([(mu_out, p) for p in range(P_mu)]
                + [(sg_out, p) for p in range(P_mu)]
                + [(al_out, p) for p in range(P_al)])
        tasks = [(pp, j) for j in range(n_chunks) for pp in range(n_planes)]

        def start_gather(t):
            pp, j = tasks[t]
            b = t % _NBUF
            return pltpu.async_copy(
                spm.at[idx_v.at[j, pp]], buf_v.at[b], sem_g.at[b])

        def start_write(t):
            pp, j = tasks[t]
            out, p = outs[pp]
            b = t % _NBUF
            base = (wid * n_chunks + j) * _CHUNK
            return pltpu.async_copy(
                buf_v.at[b], out.at[p, pl.ds(base, _CHUNK)], sem_w.at[b])

        n_tasks = len(tasks)
        gather_cps = [None] * _NBUF
        write_cps = [None] * _NBUF
        for t in range(min(_NBUF - 1, n_tasks)):
            gather_cps[t % _NBUF] = start_gather(t)
        for t in range(n_tasks):
            b = t % _NBUF
            gather_cps[b].wait()
            write_cps[b] = start_write(t)
            nxt = t + _NBUF - 1
            if nxt < n_tasks:
                nb = nxt % _NBUF
                if write_cps[nb] is not None:
                    write_cps[nb].wait()
                    write_cps[nb] = None
                gather_cps[nb] = start_gather(nxt)
        for b in range(_NBUF):
            if write_cps[b] is not None:
                write_cps[b].wait()

    return gather_kernel


def kernel(category_ids, mu, sigma, alpha):
    B = category_ids.shape[0]
    C, K, P_mu = mu.shape
    P_al = alpha.shape[2]
    n_planes = 2 * P_mu + P_al
    ids = category_ids.astype(jnp.int32)
    offs = jnp.arange(n_planes, dtype=jnp.int32) * C
    ids7 = ids[:, None] + offs[None, :]
    ids7 = jnp.transpose(ids7.reshape(B // _CHUNK, _CHUNK, n_planes),
                         (0, 2, 1))
    mu_p = jnp.transpose(mu, (2, 0, 1)).reshape(P_mu * C, K)
    sg_p = jnp.transpose(sigma, (2, 0, 1)).reshape(P_mu * C, K)
    al_p = jnp.transpose(alpha, (2, 0, 1)).reshape(P_al * C, K)
    f = _build(B, C, K, P_mu, P_al)
    mu_o, sg_o, al_o = f(ids7, mu_p, sg_p, al_p)
    return (jnp.transpose(mu_o, (1, 2, 0)),
            jnp.transpose(sg_o, (1, 2, 0)),
            jnp.transpose(al_o, (1, 2, 0)))
